# baseline (speedup 1.0000x reference)
import functools

import jax
import jax.numpy as jnp
from jax import lax
from jax.experimental import pallas as pl
from jax.experimental.pallas import tpu as pltpu
from jax.experimental.pallas import tpu_sc as plsc

_NB = 4
_NV = 10000
_DY = 128
_DE = 64
_NW = 32
_ROWS = _NB * _NV
_RPW = _ROWS // _NW
_NCHUNK = 10
_C = _RPW // _NCHUNK
_NSLOT = 3
_SPAD = 10240


@functools.lru_cache(maxsize=1)
def _build():
    mesh = plsc.VectorSubcoreMesh(core_axis_name="c", subcore_axis_name="s")

    @functools.partial(
        pl.kernel,
        out_type=(
            jax.ShapeDtypeStruct((_ROWS + 128, _DY), jnp.float32),
            jax.ShapeDtypeStruct((_ROWS + 128, _DY), jnp.int32),
        ),
        mesh=mesh,
        compiler_params=pltpu.CompilerParams(
            use_tc_tiling_on_sc=False, needs_layout_passes=False
        ),
        scratch_types=[
            pltpu.VMEM((_NCHUNK, 128), jnp.int32),
            pltpu.VMEM((_SPAD,), jnp.int32),
            pltpu.VMEM((_NSLOT, 128, _DY), jnp.float32),
            pltpu.VMEM((_NSLOT, 128, _DY), jnp.int32),
            [pltpu.SemaphoreType.DMA] * _NSLOT,
            [pltpu.SemaphoreType.DMA] * _NSLOT,
            [pltpu.SemaphoreType.DMA] * _NSLOT,
            [pltpu.SemaphoreType.DMA] * _NSLOT,
        ],
    )
    def _shuffle(
        y_hbm, ef_hbm, idx_hbm, s_hbm,
        y_out, ef_out,
        idx_v, s_v, ybuf, efbuf,
        gsy, ssy, gse, sse,
    ):
        wid = lax.axis_index("s") * 2 + lax.axis_index("c")
        row0 = wid * _RPW
        pltpu.sync_copy(idx_hbm.at[wid], idx_v)
        pltpu.sync_copy(s_hbm, s_v)

        def _gather(c):
            k = c % _NSLOT
            ix = idx_v.at[c]
            return (
                pltpu.async_copy(y_hbm.at[ix], ybuf.at[k], gsy[k]),
                pltpu.async_copy(ef_hbm.at[ix], efbuf.at[k], gse[k]),
            )

        def _scatter(c):
            k = c % _NSLOT
            dst = pl.ds(row0 + c * _C, 128)
            return (
                pltpu.async_copy(ybuf.at[k], y_out.at[dst], ssy[k]),
                pltpu.async_copy(efbuf.at[k], ef_out.at[dst], sse[k]),
            )

        g = [None] * _NCHUNK
        sc = [None] * _NCHUNK
        g[0] = _gather(0)
        g[1] = _gather(1)
        for c in range(_NCHUNK):
            k = c % _NSLOT
            g[c][1].wait()

            def _remap(r, _):
                for q in range(_DE // 16):
                    col = pl.ds(q * 16, 16)
                    ev = efbuf[k, r, col]
                    efbuf[k, r, col] = plsc.load_gather(s_v, [ev])
                return 0

            lax.fori_loop(0, 128, _remap, 0)
            g[c][0].wait()
            sc[c] = _scatter(c)
            if c + 2 < _NCHUNK:
                if c >= 1:
                    for d in sc[c - 1]:
                        d.wait()
                g[c + 2] = _gather(c + 2)
        for c in (_NCHUNK - 3, _NCHUNK - 2, _NCHUNK - 1):
            for d in sc[c]:
                d.wait()

    return _shuffle


def _stage_s():
    return jax.random.permutation(
        jax.random.key(42), jnp.arange(_NV, dtype=jnp.int32)
    )


def kernel(y, e, f):
    s = _stage_s()
    s_pad = jnp.pad(s, (0, _SPAD - _NV))
    flat = (
        jnp.arange(_NB, dtype=jnp.int32)[:, None] * _NV + s[None, :]
    ).reshape(_ROWS)
    ext = jnp.concatenate([flat, jnp.full((128,), flat[-1], jnp.int32)])
    pos = (
        jnp.arange(_NW, dtype=jnp.int32)[:, None, None] * _RPW
        + jnp.arange(_NCHUNK, dtype=jnp.int32)[None, :, None] * _C
        + jnp.arange(128, dtype=jnp.int32)[None, None, :]
    )
    idx = ext[pos]
    ef = jnp.concatenate(
        [
            e.reshape(_ROWS, _DE),
            jax.lax.bitcast_convert_type(f.reshape(_ROWS, _DE), jnp.int32),
        ],
        axis=1,
    )
    y2, ef2 = _build()(y.reshape(_ROWS, _DY), ef, idx, s_pad)
    return (
        y2[:_ROWS].reshape(_NB, _NV, _DY),
        ef2[:_ROWS, :_DE].reshape(_NB, _NV, 4, 16),
        jax.lax.bitcast_convert_type(ef2[:_ROWS, _DE:], jnp.float32).reshape(
            _NB, _NV, 4, 16
        ),
        s,
        s,
    )


# speedup vs baseline: 1.0527x; 1.0527x over previous
import functools

import jax
import jax.numpy as jnp
from jax import lax
from jax.experimental import pallas as pl
from jax.experimental.pallas import tpu as pltpu
from jax.experimental.pallas import tpu_sc as plsc

_NB = 4
_NV = 10000
_DY = 128
_DE = 64
_NW = 32
_ROWS = _NB * _NV
_RPW = _ROWS // _NW
_NCHUNK = 10
_C = _RPW // _NCHUNK
_NSLOT = 3


def _mesh():
    return plsc.VectorSubcoreMesh(core_axis_name="c", subcore_axis_name="s")


@functools.lru_cache(maxsize=1)
def _build_y():
    @functools.partial(
        pl.kernel,
        out_type=jax.ShapeDtypeStruct((_ROWS, _DY), jnp.float32),
        mesh=_mesh(),
        compiler_params=pltpu.CompilerParams(
            use_tc_tiling_on_sc=False, needs_layout_passes=False
        ),
        scratch_types=[
            pltpu.VMEM((_NCHUNK, _C), jnp.int32),
            pltpu.VMEM((_NSLOT, _C, _DY), jnp.float32),
            [pltpu.SemaphoreType.DMA] * _NSLOT,
            [pltpu.SemaphoreType.DMA] * _NSLOT,
        ],
    )
    def _shuffle_y(y_hbm, idx_hbm, y_out, idx_v, ybuf, gsem, ssem):
        wid = lax.axis_index("s") * 2 + lax.axis_index("c")
        row0 = wid * _RPW
        pltpu.sync_copy(idx_hbm.at[wid], idx_v)

        def _gather(c):
            k = c % _NSLOT
            return pltpu.async_copy(y_hbm.at[idx_v.at[c]], ybuf.at[k], gsem[k])

        def _scatter(c):
            k = c % _NSLOT
            dst = row0 + c * _C
            return pltpu.async_copy(ybuf.at[k], y_out.at[pl.ds(dst, _C)], ssem[k])

        g = [None] * _NCHUNK
        sc = [None] * _NCHUNK
        g[0] = _gather(0)
        g[1] = _gather(1)
        for c in range(_NCHUNK):
            g[c].wait()
            sc[c] = _scatter(c)
            if c + 2 < _NCHUNK:
                if c >= 1:
                    sc[c - 1].wait()
                g[c + 2] = _gather(c + 2)
        for c in (_NCHUNK - 3, _NCHUNK - 2, _NCHUNK - 1):
            sc[c].wait()

    return _shuffle_y


@functools.lru_cache(maxsize=1)
def _build_ef():
    @functools.partial(
        pl.kernel,
        out_type=jax.ShapeDtypeStruct((_ROWS, _DY), jnp.int32),
        mesh=_mesh(),
        compiler_params=pltpu.CompilerParams(
            use_tc_tiling_on_sc=False, needs_layout_passes=False
        ),
        scratch_types=[
            pltpu.VMEM((_NCHUNK, _C), jnp.int32),
            pltpu.VMEM((_NV,), jnp.int32),
            pltpu.VMEM((_NSLOT, _C, _DY), jnp.int32),
            [pltpu.SemaphoreType.DMA] * _NSLOT,
            [pltpu.SemaphoreType.DMA] * _NSLOT,
        ],
    )
    def _shuffle_ef(ef_hbm, idx_hbm, s_hbm, ef_out, idx_v, s_v, efbuf, gsem, ssem):
        wid = lax.axis_index("s") * 2 + lax.axis_index("c")
        row0 = wid * _RPW
        pltpu.sync_copy(idx_hbm.at[wid], idx_v)
        pltpu.sync_copy(s_hbm, s_v)

        def _gather(c):
            k = c % _NSLOT
            return pltpu.async_copy(ef_hbm.at[idx_v.at[c]], efbuf.at[k], gsem[k])

        def _scatter(c):
            k = c % _NSLOT
            dst = row0 + c * _C
            return pltpu.async_copy(efbuf.at[k], ef_out.at[pl.ds(dst, _C)], ssem[k])

        g = [None] * _NCHUNK
        sc = [None] * _NCHUNK
        g[0] = _gather(0)
        g[1] = _gather(1)
        for c in range(_NCHUNK):
            k = c % _NSLOT
            g[c].wait()

            def _remap(r, _):
                for q in range(_DE // 16):
                    col = pl.ds(q * 16, 16)
                    ev = efbuf[k, r, col]
                    efbuf[k, r, col] = plsc.load_gather(s_v, [ev])
                return 0

            lax.fori_loop(0, _C, _remap, 0)
            sc[c] = _scatter(c)
            if c + 2 < _NCHUNK:
                if c >= 1:
                    sc[c - 1].wait()
                g[c + 2] = _gather(c + 2)
        for c in (_NCHUNK - 3, _NCHUNK - 2, _NCHUNK - 1):
            sc[c].wait()

    return _shuffle_ef


def _stage_s():
    return jax.random.permutation(
        jax.random.key(42), jnp.arange(_NV, dtype=jnp.int32)
    )


def kernel(y, e, f):
    s = _stage_s()
    idx = (jnp.arange(_NB, dtype=jnp.int32)[:, None] * _NV + s[None, :]).reshape(
        _NW, _NCHUNK, _C
    )
    ef = jnp.concatenate(
        [
            e.reshape(_ROWS, _DE),
            jax.lax.bitcast_convert_type(f.reshape(_ROWS, _DE), jnp.int32),
        ],
        axis=1,
    )
    y2 = _build_y()(y.reshape(_ROWS, _DY), idx)
    ef2 = _build_ef()(ef, idx, s)
    return (
        y2.reshape(_NB, _NV, _DY),
        ef2[:, :_DE].reshape(_NB, _NV, 4, 16),
        jax.lax.bitcast_convert_type(ef2[:, _DE:], jnp.float32).reshape(
            _NB, _NV, 4, 16
        ),
        s,
        s,
    )


# speedup vs baseline: 1.2273x; 1.1658x over previous
import functools

import jax
import jax.numpy as jnp
from jax import lax
from jax.experimental import pallas as pl
from jax.experimental.pallas import tpu as pltpu
from jax.experimental.pallas import tpu_sc as plsc

_NB = 4
_NV = 10000
_DY = 128
_NR, _ND = 4, 16
_NW = 32
_ROWS = _NB * _NV
_RPW = _ROWS // _NW
_NCHUNK = 10
_C = _RPW // _NCHUNK
_NSLOT = 3

_EFROWS = _NB * _NR * _ND
_STRIP = _EFROWS // _NW
_VC = 1024
_NVC = (_NV + _VC - 1) // _VC


@functools.lru_cache(maxsize=1)
def _build_y():
    mesh = plsc.VectorSubcoreMesh(core_axis_name="c", subcore_axis_name="s")

    @functools.partial(
        pl.kernel,
        out_type=jax.ShapeDtypeStruct((_ROWS, _DY), jnp.float32),
        mesh=mesh,
        compiler_params=pltpu.CompilerParams(
            use_tc_tiling_on_sc=False, needs_layout_passes=False
        ),
        scratch_types=[
            pltpu.VMEM((_NCHUNK, _C), jnp.int32),
            pltpu.VMEM((_NSLOT, _C, _DY), jnp.float32),
            [pltpu.SemaphoreType.DMA] * _NSLOT,
            [pltpu.SemaphoreType.DMA] * _NSLOT,
        ],
    )
    def _shuffle_y(y_hbm, idx_hbm, y_out, idx_v, ybuf, gsem, ssem):
        wid = lax.axis_index("s") * 2 + lax.axis_index("c")
        row0 = wid * _RPW
        pltpu.sync_copy(idx_hbm.at[wid], idx_v)

        def _gather(c):
            k = c % _NSLOT
            return pltpu.async_copy(y_hbm.at[idx_v.at[c]], ybuf.at[k], gsem[k])

        def _scatter(c):
            k = c % _NSLOT
            dst = row0 + c * _C
            return pltpu.async_copy(ybuf.at[k], y_out.at[pl.ds(dst, _C)], ssem[k])

        g = [None] * _NCHUNK
        sc = [None] * _NCHUNK
        g[0] = _gather(0)
        g[1] = _gather(1)
        for c in range(_NCHUNK):
            g[c].wait()
            sc[c] = _scatter(c)
            if c + 2 < _NCHUNK:
                if c >= 1:
                    sc[c - 1].wait()
                g[c + 2] = _gather(c + 2)
        for c in (_NCHUNK - 3, _NCHUNK - 2, _NCHUNK - 1):
            sc[c].wait()

    return _shuffle_y


@functools.lru_cache(maxsize=1)
def _build_ef():
    mesh = plsc.VectorSubcoreMesh(core_axis_name="c", subcore_axis_name="s")

    @functools.partial(
        pl.kernel,
        out_type=(
            jax.ShapeDtypeStruct((_EFROWS, _NV), jnp.int32),
            jax.ShapeDtypeStruct((_EFROWS, _NV), jnp.int32),
        ),
        mesh=mesh,
        compiler_params=pltpu.CompilerParams(
            use_tc_tiling_on_sc=False, needs_layout_passes=False
        ),
        scratch_types=[
            pltpu.VMEM((_NV,), jnp.int32),
            pltpu.VMEM((_STRIP, _NV), jnp.int32),
            pltpu.VMEM((2, _STRIP, _VC), jnp.int32),
            pltpu.SemaphoreType.DMA,
            [pltpu.SemaphoreType.DMA] * 2,
        ],
    )
    def _shuffle_ef(e_hbm, f_hbm, s_hbm, e_out, f_out, s_v, strip, obuf, gsem, ssem):
        wid = lax.axis_index("s") * 2 + lax.axis_index("c")
        r0 = wid * _STRIP
        pltpu.sync_copy(s_hbm, s_v)
        sc_prev = [None, None]

        def _do_array(src, dst, remap):
            pltpu.async_copy(src.at[pl.ds(r0, _STRIP)], strip, gsem).wait()
            for vc in range(_NVC):
                vbase = vc * _VC
                n = min(_VC, _NV - vbase)
                k = vc % 2
                if sc_prev[k] is not None:
                    sc_prev[k].wait()

                def _vec(j, _):
                    ixv = s_v[pl.ds(vbase + j * 16, 16)]
                    for r in range(_STRIP):
                        g = plsc.load_gather(strip.at[r], [ixv])
                        if remap:
                            g = plsc.load_gather(s_v, [g])
                        obuf[k, r, pl.ds(j * 16, 16)] = g
                    return 0

                lax.fori_loop(0, n // 16, _vec, 0)
                sc_prev[k] = pltpu.async_copy(
                    obuf.at[k, slice(None), pl.ds(0, n)],
                    dst.at[pl.ds(r0, _STRIP), pl.ds(vbase, n)],
                    ssem[k],
                )

        _do_array(e_hbm, e_out, True)
        _do_array(f_hbm, f_out, False)
        for d in sc_prev:
            if d is not None:
                d.wait()

    return _shuffle_ef


def _stage_s():
    return jax.random.permutation(
        jax.random.key(42), jnp.arange(_NV, dtype=jnp.int32)
    )


def kernel(y, e, f):
    s = _stage_s()
    idx = (jnp.arange(_NB, dtype=jnp.int32)[:, None] * _NV + s[None, :]).reshape(
        _NW, _NCHUNK, _C
    )
    y2 = _build_y()(y.reshape(_ROWS, _DY), idx).reshape(_NB, _NV, _DY)

    e2d = jnp.transpose(e, (0, 2, 3, 1)).reshape(_EFROWS, _NV)
    f2d = jax.lax.bitcast_convert_type(
        jnp.transpose(f, (0, 2, 3, 1)).reshape(_EFROWS, _NV), jnp.int32
    )
    e2o, f2o = _build_ef()(e2d, f2d, s)
    e2 = jnp.transpose(e2o.reshape(_NB, _NR, _ND, _NV), (0, 3, 1, 2))
    f2 = jnp.transpose(
        jax.lax.bitcast_convert_type(f2o, jnp.float32).reshape(_NB, _NR, _ND, _NV),
        (0, 3, 1, 2),
    )
    return (y2, e2, f2, s, s)
